# Initial kernel scaffold; baseline (speedup 1.0000x reference)
import jax
import jax.numpy as jnp
from jax.experimental import pallas as pl


def kernel(x, batch, node_attr, edge_src, edge_dst, wstatic, W_lin, embed_table, h, fc1_w, fc1_b, lin1_w):
    raise NotImplementedError("write your pallas kernel here")


# speedup vs baseline: 2.2600x; 2.2600x over previous
import functools

import jax
import jax.numpy as jnp
from jax.experimental import pallas as pl
from jax.experimental.pallas import tpu as pltpu

N = 10000
E = 160000
DL = 128
FD = 5 * DL

BE = 640


def _edge_mlp_body(ysrc, ydst, nsrc, ndst, wst, m1, m2, wrow, brow, lin1t,
                   adst, asrc):
    pre = (jnp.dot(nsrc[...], m1[...], preferred_element_type=jnp.float32)
           + jnp.dot(ndst[...], m2[...], preferred_element_type=jnp.float32)
           + wst[...] * wrow[...] + brow[...])
    W = pre * jax.nn.sigmoid(pre)
    ys = ysrc[...]
    yd = ydst[...]
    g = W * (ys - yd)
    a = W * (ys + yd) * 0.5
    dxe = jnp.concatenate([g, a, g * a, g * g, a * a], axis=1)
    z = jnp.tanh(dxe)
    lt = lin1t[...]
    t = jnp.dot(z.astype(jnp.bfloat16), lt, preferred_element_type=jnp.float32)
    t = t - jnp.mean(t, axis=1, keepdims=True)
    t = t * jax.lax.rsqrt(jnp.sum(t * t, axis=1, keepdims=True) + 1e-3)
    z2 = jnp.tanh(t)
    t2 = jnp.dot(z2.astype(jnp.bfloat16), lt, preferred_element_type=jnp.float32)
    z3 = jnp.tanh(t2)
    p = W * z3[:, :DL]
    q = (0.5 * W) * (z3[:, DL:2 * DL] + z3[:, 2 * DL:3 * DL]
                     + z3[:, 3 * DL:4 * DL] + z3[:, 4 * DL:])
    adst[...] = p + q
    asrc[...] = q - p


def _edge_mlp(ysrc, ydst, nsrc, ndst, wst, m1, m2, wrow, brow, lin1t):
    nblk = E // BE
    eb = lambda e: (e, 0)
    wb = lambda e: (0, 0)
    return pl.pallas_call(
        _edge_mlp_body,
        grid=(nblk,),
        in_specs=[
            pl.BlockSpec((BE, DL), eb),
            pl.BlockSpec((BE, DL), eb),
            pl.BlockSpec((BE, 16), eb),
            pl.BlockSpec((BE, 16), eb),
            pl.BlockSpec((BE, 1), eb),
            pl.BlockSpec((16, DL), wb),
            pl.BlockSpec((16, DL), wb),
            pl.BlockSpec((1, DL), wb),
            pl.BlockSpec((1, DL), wb),
            pl.BlockSpec((FD, FD), wb),
        ],
        out_specs=[
            pl.BlockSpec((BE, DL), eb),
            pl.BlockSpec((BE, DL), eb),
        ],
        out_shape=[
            jax.ShapeDtypeStruct((E, DL), jnp.float32),
            jax.ShapeDtypeStruct((E, DL), jnp.float32),
        ],
    )(ysrc, ydst, nsrc, ndst, wst, m1, m2, wrow, brow, lin1t)


def _uplift_body(x, wlt, o):
    o[...] = jnp.dot(x[...], wlt[...], preferred_element_type=jnp.float32)


def _uplift(x, wlt):
    bn = 2000
    return pl.pallas_call(
        _uplift_body,
        grid=(N // bn,),
        in_specs=[
            pl.BlockSpec((bn, x.shape[1]), lambda i: (i, 0)),
            pl.BlockSpec(wlt.shape, lambda i: (0, 0)),
        ],
        out_specs=pl.BlockSpec((bn, DL), lambda i: (i, 0)),
        out_shape=jax.ShapeDtypeStruct((N, DL), jnp.float32),
    )(x, wlt)


def _update_body(y, yold, s1, s2, dt, o):
    o[...] = 2.0 * y[...] - yold[...] - dt[...] * (s1[...] + s2[...])


def _update(y, yold, s1, s2, dt):
    bn = 2000
    return pl.pallas_call(
        _update_body,
        grid=(N // bn,),
        in_specs=[
            pl.BlockSpec((bn, DL), lambda i: (i, 0)),
            pl.BlockSpec((bn, DL), lambda i: (i, 0)),
            pl.BlockSpec((bn, DL), lambda i: (i, 0)),
            pl.BlockSpec((bn, DL), lambda i: (i, 0)),
            pl.BlockSpec((1, 1), lambda i: (0, 0)),
        ],
        out_specs=pl.BlockSpec((bn, DL), lambda i: (i, 0)),
        out_shape=jax.ShapeDtypeStruct((N, DL), jnp.float32),
    )(y, yold, s1, s2, dt)


def _project_body(y, wl, o):
    o[...] = jnp.dot(y[...], wl[...], preferred_element_type=jnp.float32)


def _project(y, wlpad):
    bn = 2000
    return pl.pallas_call(
        _project_body,
        grid=(N // bn,),
        in_specs=[
            pl.BlockSpec((bn, DL), lambda i: (i, 0)),
            pl.BlockSpec((DL, DL), lambda i: (0, 0)),
        ],
        out_specs=pl.BlockSpec((bn, DL), lambda i: (i, 0)),
        out_shape=jax.ShapeDtypeStruct((N, DL), jnp.float32),
    )(y, wlpad)


def kernel(x, batch, node_attr, edge_src, edge_dst, wstatic, W_lin,
           embed_table, h, fc1_w, fc1_b, lin1_w):
    L = fc1_w.shape[0]
    emb = embed_table.shape[1]

    na = embed_table[jnp.min(node_attr, axis=-1)]
    na_pad = jnp.pad(na, ((0, 0), (0, 16 - emb)))
    nsrc = na_pad[edge_src]
    ndst = na_pad[edge_dst]
    wst = wstatic[:, None]

    dts = jnp.clip(h * h, 1e-4, 0.1)

    y = _uplift(x, W_lin.T)
    y_old = y

    for i in range(L):
        f1t = fc1_w[i].T
        m1 = jnp.zeros((16, DL), jnp.float32).at[:emb].set(f1t[:emb])
        m2 = jnp.zeros((16, DL), jnp.float32).at[:emb].set(f1t[emb:2 * emb])
        wrow = f1t[2 * emb:2 * emb + 1]
        brow = fc1_b[i][None, :]
        lin1t = lin1_w[i].T.astype(jnp.bfloat16)

        ysrc = y[edge_src]
        ydst = y[edge_dst]
        adst, asrc = _edge_mlp(ysrc, ydst, nsrc, ndst, wst,
                               m1, m2, wrow, brow, lin1t)
        s1 = jax.ops.segment_sum(adst, edge_dst, num_segments=N)
        s2 = jax.ops.segment_sum(asrc, edge_src, num_segments=N)
        dt = dts[i].reshape(1, 1)
        y_new = _update(y, y_old, s1, s2, dt)
        y_old = y
        y = y_new

    wlpad = jnp.zeros((DL, DL), jnp.float32).at[:, :W_lin.shape[1]].set(W_lin)
    x_out = _project(y, wlpad)[:, :W_lin.shape[1]]
    reg = jnp.asarray(0.0, dtype=jnp.float32)
    return x_out, reg


# speedup vs baseline: 2.8361x; 1.2549x over previous
import functools

import jax
import jax.numpy as jnp
from jax.experimental import pallas as pl
from jax.experimental.pallas import tpu as pltpu
from jax.experimental.pallas import tpu_sc as plsc

N = 10000
E = 160000
DL = 128
FD = 5 * DL

BE = 640
GW = 128
SW = 128
NPAD = 10240
ZR = 80


def _sc_mesh():
    return plsc.VectorSubcoreMesh(core_axis_name="c", subcore_axis_name="s")


def _sc_gather(table, idx):
    M = idx.shape[0]
    C = table.shape[1]
    idx2 = idx.reshape(1, M)

    @functools.partial(
        pl.kernel,
        out_type=jax.ShapeDtypeStruct((M, C), table.dtype),
        mesh=_sc_mesh(),
    )
    def k(tab_hbm, i_hbm, o_hbm):
        def body(i_vmem, o_vmem):
            pltpu.sync_copy(tab_hbm.at[i_vmem.at[0]], o_vmem)

        pltpu.emit_pipeline(
            body,
            grid=(M // GW,),
            in_specs=[pl.BlockSpec((1, GW), lambda i: (0, i))],
            out_specs=[pl.BlockSpec((GW, C), lambda i: (i, 0))],
            core_axis_name=("c", "s"),
            dimension_semantics=(pltpu.PARALLEL,),
        )(i_hbm, o_hbm)

    return k(table, idx2)


def _sc_scatter2(vals, idx2):
    C = vals.shape[2]
    idx3 = idx2.reshape(2, 1, E)
    rows_per = NPAD // 16

    @functools.partial(
        pl.kernel,
        out_type=jax.ShapeDtypeStruct((2, NPAD, C), vals.dtype),
        mesh=_sc_mesh(),
        scratch_types=[
            pltpu.VMEM_SHARED((NPAD, C), vals.dtype),
            pltpu.VMEM((ZR, C), vals.dtype),
        ],
    )
    def k(v_hbm, i_hbm, o_hbm, acc, zbuf):
        c = jax.lax.axis_index("c")
        s = jax.lax.axis_index("s")

        @pl.loop(0, ZR)
        def _(r):
            @pl.loop(0, C, step=16)
            def _(j):
                zbuf.at[pl.ds(r, 1), pl.ds(j, 16)][...] = jnp.zeros(
                    (1, 16), vals.dtype)

        @pl.loop(0, rows_per, step=ZR)
        def _(r0):
            pltpu.sync_copy(zbuf, acc.at[pl.ds(s * rows_per + r0, ZR)])

        plsc.subcore_barrier()

        def body(v_vmem, i_vmem):
            pltpu.sync_copy(v_vmem, acc.at[i_vmem.at[0]], add=True)

        pltpu.emit_pipeline(
            body,
            grid=(E // SW,),
            in_specs=[
                pl.BlockSpec((SW, C), lambda i: (i, 0)),
                pl.BlockSpec((1, SW), lambda i: (0, i)),
            ],
            out_specs=[],
            core_axis_name=("s",),
            dimension_semantics=(pltpu.PARALLEL,),
        )(v_hbm.at[c], i_hbm.at[c])

        plsc.subcore_barrier()

        @pl.loop(0, rows_per, step=ZR)
        def _(r0):
            pltpu.sync_copy(acc.at[pl.ds(s * rows_per + r0, ZR)],
                            o_hbm.at[c].at[pl.ds(s * rows_per + r0, ZR)])

    return k(vals, idx3)


def _edge_mlp_body(yall, wpre, wst, wrow, lin1t, out):
    ys = yall[0]
    yd = yall[1]
    pre = wpre[...] + wst[...] * wrow[...]
    W = pre * jax.nn.sigmoid(pre)
    g = W * (ys - yd)
    a = W * (ys + yd) * 0.5
    dxe = jnp.concatenate([g, a, g * a, g * g, a * a], axis=1)
    z = jnp.tanh(dxe)
    lt = lin1t[...]
    t = jnp.dot(z.astype(jnp.bfloat16), lt, preferred_element_type=jnp.float32)
    t = t - jnp.mean(t, axis=1, keepdims=True)
    t = t * jax.lax.rsqrt(jnp.sum(t * t, axis=1, keepdims=True) + 1e-3)
    z2 = jnp.tanh(t)
    t2 = jnp.dot(z2.astype(jnp.bfloat16), lt, preferred_element_type=jnp.float32)
    z3 = jnp.tanh(t2)
    p = W * z3[:, :DL]
    q = (0.5 * W) * (z3[:, DL:2 * DL] + z3[:, 2 * DL:3 * DL]
                     + z3[:, 3 * DL:4 * DL] + z3[:, 4 * DL:])
    out[0] = p + q
    out[1] = q - p


def _edge_mlp(yall3, wpre, wst, wrow, lin1t):
    nblk = E // BE
    e3 = lambda e: (0, e, 0)
    wb = lambda e: (0, 0)
    return pl.pallas_call(
        _edge_mlp_body,
        grid=(nblk,),
        in_specs=[
            pl.BlockSpec((2, BE, DL), e3),
            pl.BlockSpec((BE, DL), lambda e: (e, 0)),
            pl.BlockSpec((BE, 1), lambda e: (e, 0)),
            pl.BlockSpec((1, DL), wb),
            pl.BlockSpec((FD, FD), wb),
        ],
        out_specs=pl.BlockSpec((2, BE, DL), e3),
        out_shape=jax.ShapeDtypeStruct((2, E, DL), jnp.float32),
    )(yall3, wpre, wst, wrow, lin1t)


def _uplift_body(x, wlt, o):
    o[...] = jnp.dot(x[...], wlt[...], preferred_element_type=jnp.float32)


def _uplift(x, wlt):
    bn = 2000
    return pl.pallas_call(
        _uplift_body,
        grid=(N // bn,),
        in_specs=[
            pl.BlockSpec((bn, x.shape[1]), lambda i: (i, 0)),
            pl.BlockSpec(wlt.shape, lambda i: (0, 0)),
        ],
        out_specs=pl.BlockSpec((bn, DL), lambda i: (i, 0)),
        out_shape=jax.ShapeDtypeStruct((N, DL), jnp.float32),
    )(x, wlt)


def _update_body(y, yold, sp, dt, o):
    o[...] = 2.0 * y[...] - yold[...] - dt[...] * (sp[0] + sp[1])


def _update(y, yold, sp, dt):
    bn = 2000
    return pl.pallas_call(
        _update_body,
        grid=(N // bn,),
        in_specs=[
            pl.BlockSpec((bn, DL), lambda i: (i, 0)),
            pl.BlockSpec((bn, DL), lambda i: (i, 0)),
            pl.BlockSpec((2, bn, DL), lambda i: (0, i, 0)),
            pl.BlockSpec((1, 1), lambda i: (0, 0)),
        ],
        out_specs=pl.BlockSpec((bn, DL), lambda i: (i, 0)),
        out_shape=jax.ShapeDtypeStruct((N, DL), jnp.float32),
    )(y, yold, sp, dt)


def _project_body(y, wl, o):
    o[...] = jnp.dot(y[...], wl[...], preferred_element_type=jnp.float32)


def _project(y, wlpad):
    bn = 2000
    return pl.pallas_call(
        _project_body,
        grid=(N // bn,),
        in_specs=[
            pl.BlockSpec((bn, DL), lambda i: (i, 0)),
            pl.BlockSpec((DL, DL), lambda i: (0, 0)),
        ],
        out_specs=pl.BlockSpec((bn, DL), lambda i: (i, 0)),
        out_shape=jax.ShapeDtypeStruct((N, DL), jnp.float32),
    )(y, wlpad)


def kernel(x, batch, node_attr, edge_src, edge_dst, wstatic, W_lin,
           embed_table, h, fc1_w, fc1_b, lin1_w):
    L = fc1_w.shape[0]
    emb = embed_table.shape[1]

    nt = embed_table.shape[0]
    nidx = jnp.min(node_attr, axis=-1).astype(jnp.int32)
    pidx = nidx[edge_src] * nt + nidx[edge_dst]

    idx_all = jnp.concatenate([edge_src, edge_dst]).astype(jnp.int32)
    idx2 = jnp.stack([edge_dst, edge_src]).astype(jnp.int32)
    wst = wstatic[:, None]

    dts = jnp.clip(h * h, 1e-4, 0.1)

    y = _uplift(x, W_lin.T)
    y_old = y

    for i in range(L):
        f1t = fc1_w[i].T
        g1 = embed_table @ f1t[:emb]
        g2 = embed_table @ f1t[emb:2 * emb]
        gpair = (g1[:, None, :] + g2[None, :, :]
                 + fc1_b[i][None, None, :]).reshape(nt * nt, DL)
        wrow = f1t[2 * emb:2 * emb + 1]
        lin1t = lin1_w[i].T.astype(jnp.bfloat16)

        wpre = _sc_gather(gpair, pidx)
        yall3 = _sc_gather(y, idx_all).reshape(2, E, DL)
        av = _edge_mlp(yall3, wpre, wst, wrow, lin1t)
        sp = _sc_scatter2(av, idx2)
        dt = dts[i].reshape(1, 1)
        y_new = _update(y, y_old, sp, dt)
        y_old = y
        y = y_new

    wlpad = jnp.zeros((DL, DL), jnp.float32).at[:, :W_lin.shape[1]].set(W_lin)
    x_out = _project(y, wlpad)[:, :W_lin.shape[1]]
    reg = jnp.asarray(0.0, dtype=jnp.float32)
    return x_out, reg


# speedup vs baseline: 5.9814x; 2.1090x over previous
import dataclasses
import functools

import jax
import jax.numpy as jnp
from jax.experimental import pallas as pl
from jax.experimental.pallas import tpu as pltpu
from jax.experimental.pallas import tpu_sc as plsc

N = 10000
E = 160000
DL = 128
FD = 5 * DL

BE = 640
GW = 128
SW = 128
NPAD = 10240
ZR = 80


def _sc_mesh():
    return plsc.VectorSubcoreMesh(core_axis_name="c", subcore_axis_name="s")


def _sc_gather(table, idx):
    M = idx.shape[0]
    C = table.shape[1]
    idx2 = idx.reshape(1, M)

    @functools.partial(
        pl.kernel,
        out_type=jax.ShapeDtypeStruct((M, C), table.dtype),
        mesh=_sc_mesh(),
    )
    def k(tab_hbm, i_hbm, o_hbm):
        def body(i_vmem, o_vmem):
            pltpu.sync_copy(tab_hbm.at[i_vmem.at[0]], o_vmem)

        pltpu.emit_pipeline(
            body,
            grid=(M // GW,),
            in_specs=[pl.BlockSpec((1, GW), lambda i: (0, i))],
            out_specs=[pl.BlockSpec((GW, C), lambda i: (i, 0))],
            core_axis_name=("c", "s"),
            dimension_semantics=(pltpu.PARALLEL,),
        )(i_hbm, o_hbm)

    return k(table, idx2)


def _sc_types(nidx_pad, esrc2, edst2):
    cp = pltpu.CompilerParams()
    if "needs_layout_passes" in pltpu.CompilerParams.__dataclass_fields__:
        cp = dataclasses.replace(cp, needs_layout_passes=False)

    @functools.partial(
        pl.kernel,
        out_type=jax.ShapeDtypeStruct((2, 1, E), jnp.int32),
        mesh=_sc_mesh(),
        scratch_types=[pltpu.VMEM((NPAD,), jnp.int32)],
        compiler_params=cp,
    )
    def k(n_hbm, s_hbm, d_hbm, o_hbm, ntile):
        pltpu.sync_copy(n_hbm, ntile)

        def body(s_vmem, d_vmem, ts_vmem, td_vmem):
            @pl.loop(0, SW, step=16)
            def _(j):
                sv = s_vmem[0, pl.ds(j, 16)]
                dv = d_vmem[0, pl.ds(j, 16)]
                ts_vmem[0, pl.ds(j, 16)] = plsc.load_gather(ntile, [sv])
                td_vmem[0, pl.ds(j, 16)] = plsc.load_gather(ntile, [dv])

        pltpu.emit_pipeline(
            body,
            grid=(E // SW,),
            in_specs=[
                pl.BlockSpec((1, SW), lambda i: (0, i)),
                pl.BlockSpec((1, SW), lambda i: (0, i)),
            ],
            out_specs=[
                pl.BlockSpec((1, SW), lambda i: (0, i)),
                pl.BlockSpec((1, SW), lambda i: (0, i)),
            ],
            core_axis_name=("c", "s"),
            dimension_semantics=(pltpu.PARALLEL,),
        )(s_hbm, d_hbm, o_hbm.at[0], o_hbm.at[1])

    return k(nidx_pad, esrc2, edst2)


def _sc_scatter2(vals, idx2):
    C = vals.shape[2]
    idx3 = idx2.reshape(2, 1, E)
    rows_per = NPAD // 16

    @functools.partial(
        pl.kernel,
        out_type=jax.ShapeDtypeStruct((2, NPAD, C), vals.dtype),
        mesh=_sc_mesh(),
        scratch_types=[
            pltpu.VMEM_SHARED((NPAD, C), vals.dtype),
            pltpu.VMEM((ZR, C), vals.dtype),
        ],
    )
    def k(v_hbm, i_hbm, o_hbm, acc, zbuf):
        c = jax.lax.axis_index("c")
        s = jax.lax.axis_index("s")

        @pl.loop(0, ZR)
        def _(r):
            @pl.loop(0, C, step=16)
            def _(j):
                zbuf.at[pl.ds(r, 1), pl.ds(j, 16)][...] = jnp.zeros(
                    (1, 16), vals.dtype)

        @pl.loop(0, rows_per, step=ZR)
        def _(r0):
            pltpu.sync_copy(zbuf, acc.at[pl.ds(s * rows_per + r0, ZR)])

        plsc.subcore_barrier()

        def body(v_vmem, i_vmem):
            pltpu.sync_copy(v_vmem, acc.at[i_vmem.at[0]], add=True)

        pltpu.emit_pipeline(
            body,
            grid=(E // SW,),
            in_specs=[
                pl.BlockSpec((SW, C), lambda i: (i, 0)),
                pl.BlockSpec((1, SW), lambda i: (0, i)),
            ],
            out_specs=[],
            core_axis_name=("s",),
            dimension_semantics=(pltpu.PARALLEL,),
        )(v_hbm.at[c], i_hbm.at[c])

        plsc.subcore_barrier()

        @pl.loop(0, rows_per, step=ZR)
        def _(r0):
            pltpu.sync_copy(acc.at[pl.ds(s * rows_per + r0, ZR)],
                            o_hbm.at[c].at[pl.ds(s * rows_per + r0, ZR)])

    return k(vals, idx3)


def _edge_mlp_body(yall, ts, td, wst, g1b, g2b, wrow, lin1t, out):
    ys = yall[0]
    yd = yall[1]
    ioh = jax.lax.broadcasted_iota(jnp.int32, (1, 32), 1)
    ohs = (ts[...] == ioh).astype(jnp.float32)
    ohd = (td[...] == ioh).astype(jnp.float32)
    pre = (jnp.dot(ohs, g1b[...], preferred_element_type=jnp.float32)
           + jnp.dot(ohd, g2b[...], preferred_element_type=jnp.float32)
           + wst[...] * wrow[...])
    W = pre * jax.nn.sigmoid(pre)
    g = W * (ys - yd)
    a = W * (ys + yd) * 0.5
    dxe = jnp.concatenate([g, a, g * a, g * g, a * a], axis=1)
    z = jnp.tanh(dxe)
    lt = lin1t[...]
    t = jnp.dot(z.astype(jnp.bfloat16), lt, preferred_element_type=jnp.float32)
    t = t - jnp.mean(t, axis=1, keepdims=True)
    t = t * jax.lax.rsqrt(jnp.sum(t * t, axis=1, keepdims=True) + 1e-3)
    z2 = jnp.tanh(t)
    t2 = jnp.dot(z2.astype(jnp.bfloat16), lt, preferred_element_type=jnp.float32)
    z3 = jnp.tanh(t2)
    p = W * z3[:, :DL]
    q = (0.5 * W) * (z3[:, DL:2 * DL] + z3[:, 2 * DL:3 * DL]
                     + z3[:, 3 * DL:4 * DL] + z3[:, 4 * DL:])
    out[0] = p + q
    out[1] = q - p


def _edge_mlp(yall3, ts, td, wst, g1b, g2b, wrow, lin1t):
    nblk = E // BE
    e3 = lambda e: (0, e, 0)
    eb = lambda e: (e, 0)
    wb = lambda e: (0, 0)
    return pl.pallas_call(
        _edge_mlp_body,
        grid=(nblk,),
        in_specs=[
            pl.BlockSpec((2, BE, DL), e3),
            pl.BlockSpec((BE, 1), eb),
            pl.BlockSpec((BE, 1), eb),
            pl.BlockSpec((BE, 1), eb),
            pl.BlockSpec((32, DL), wb),
            pl.BlockSpec((32, DL), wb),
            pl.BlockSpec((1, DL), wb),
            pl.BlockSpec((FD, FD), wb),
        ],
        out_specs=pl.BlockSpec((2, BE, DL), e3),
        out_shape=jax.ShapeDtypeStruct((2, E, DL), jnp.float32),
    )(yall3, ts, td, wst, g1b, g2b, wrow, lin1t)


def _uplift_body(x, wlt, o):
    o[...] = jnp.dot(x[...], wlt[...], preferred_element_type=jnp.float32)


def _uplift(x, wlt):
    bn = 2000
    return pl.pallas_call(
        _uplift_body,
        grid=(N // bn,),
        in_specs=[
            pl.BlockSpec((bn, x.shape[1]), lambda i: (i, 0)),
            pl.BlockSpec(wlt.shape, lambda i: (0, 0)),
        ],
        out_specs=pl.BlockSpec((bn, DL), lambda i: (i, 0)),
        out_shape=jax.ShapeDtypeStruct((N, DL), jnp.float32),
    )(x, wlt)


def _update_body(y, yold, sp, dt, o):
    o[...] = 2.0 * y[...] - yold[...] - dt[...] * (sp[0] + sp[1])


def _update(y, yold, sp, dt):
    bn = 2000
    return pl.pallas_call(
        _update_body,
        grid=(N // bn,),
        in_specs=[
            pl.BlockSpec((bn, DL), lambda i: (i, 0)),
            pl.BlockSpec((bn, DL), lambda i: (i, 0)),
            pl.BlockSpec((2, bn, DL), lambda i: (0, i, 0)),
            pl.BlockSpec((1, 1), lambda i: (0, 0)),
        ],
        out_specs=pl.BlockSpec((bn, DL), lambda i: (i, 0)),
        out_shape=jax.ShapeDtypeStruct((N, DL), jnp.float32),
    )(y, yold, sp, dt)


def _project_body(y, wl, o):
    o[...] = jnp.dot(y[...], wl[...], preferred_element_type=jnp.float32)


def _project(y, wlpad):
    bn = 2000
    return pl.pallas_call(
        _project_body,
        grid=(N // bn,),
        in_specs=[
            pl.BlockSpec((bn, DL), lambda i: (i, 0)),
            pl.BlockSpec((DL, DL), lambda i: (0, 0)),
        ],
        out_specs=pl.BlockSpec((bn, DL), lambda i: (i, 0)),
        out_shape=jax.ShapeDtypeStruct((N, DL), jnp.float32),
    )(y, wlpad)


def kernel(x, batch, node_attr, edge_src, edge_dst, wstatic, W_lin,
           embed_table, h, fc1_w, fc1_b, lin1_w):
    L = fc1_w.shape[0]
    emb = embed_table.shape[1]

    nt = embed_table.shape[0]
    nidx = jnp.min(node_attr, axis=-1).astype(jnp.int32)
    nidx_pad = jnp.concatenate([nidx, jnp.zeros((NPAD - N,), jnp.int32)])
    esrc = edge_src.astype(jnp.int32)
    edst = edge_dst.astype(jnp.int32)
    tsd = _sc_types(nidx_pad, esrc.reshape(1, E), edst.reshape(1, E))
    ts = tsd[0].reshape(E, 1)
    td = tsd[1].reshape(E, 1)

    idx_all = jnp.concatenate([esrc, edst])
    idx2 = jnp.stack([edst, esrc])
    wst = wstatic[:, None]

    dts = jnp.clip(h * h, 1e-4, 0.1)

    y = _uplift(x, W_lin.T)
    y_old = y

    for i in range(L):
        f1t = fc1_w[i].T
        g1b = jnp.zeros((32, DL), jnp.float32).at[:nt].set(
            embed_table @ f1t[:emb] + fc1_b[i][None, :])
        g2b = jnp.zeros((32, DL), jnp.float32).at[:nt].set(
            embed_table @ f1t[emb:2 * emb])
        wrow = f1t[2 * emb:2 * emb + 1]
        lin1t = lin1_w[i].T.astype(jnp.bfloat16)

        yall3 = _sc_gather(y, idx_all).reshape(2, E, DL)
        av = _edge_mlp(yall3, ts, td, wst, g1b, g2b, wrow, lin1t)
        sp = _sc_scatter2(av, idx2)
        dt = dts[i].reshape(1, 1)
        y_new = _update(y, y_old, sp, dt)
        y_old = y
        y = y_new

    wlpad = jnp.zeros((DL, DL), jnp.float32).at[:, :W_lin.shape[1]].set(W_lin)
    x_out = _project(y, wlpad)[:, :W_lin.shape[1]]
    reg = jnp.asarray(0.0, dtype=jnp.float32)
    return x_out, reg


# speedup vs baseline: 6.3786x; 1.0664x over previous
import dataclasses
import functools

import jax
import jax.numpy as jnp
from jax.experimental import pallas as pl
from jax.experimental.pallas import tpu as pltpu
from jax.experimental.pallas import tpu_sc as plsc

N = 10000
E = 160000
DL = 128
FD = 5 * DL

BE = 640
GW = 128
SW = 128
NPAD = 10240
ZR = 80


def _sc_mesh():
    return plsc.VectorSubcoreMesh(core_axis_name="c", subcore_axis_name="s")


def _sc_gather(table, idx):
    M = idx.shape[0]
    C = table.shape[1]
    idx2 = idx.reshape(1, M)

    @functools.partial(
        pl.kernel,
        out_type=jax.ShapeDtypeStruct((M, C), table.dtype),
        mesh=_sc_mesh(),
    )
    def k(tab_hbm, i_hbm, o_hbm):
        def body(i_vmem, o_vmem):
            pltpu.sync_copy(tab_hbm.at[i_vmem.at[0]], o_vmem)

        pltpu.emit_pipeline(
            body,
            grid=(M // GW,),
            in_specs=[pl.BlockSpec((1, GW), lambda i: (0, i))],
            out_specs=[pl.BlockSpec((GW, C), lambda i: (i, 0))],
            core_axis_name=("c", "s"),
            dimension_semantics=(pltpu.PARALLEL,),
        )(i_hbm, o_hbm)

    return k(table, idx2)


def _sc_types(nidx_pad, esrc2, edst2):
    cp = pltpu.CompilerParams()
    if "needs_layout_passes" in pltpu.CompilerParams.__dataclass_fields__:
        cp = dataclasses.replace(cp, needs_layout_passes=False)

    @functools.partial(
        pl.kernel,
        out_type=jax.ShapeDtypeStruct((2, 1, E), jnp.int32),
        mesh=_sc_mesh(),
        scratch_types=[pltpu.VMEM((NPAD,), jnp.int32)],
        compiler_params=cp,
    )
    def k(n_hbm, s_hbm, d_hbm, o_hbm, ntile):
        pltpu.sync_copy(n_hbm, ntile)

        def body(s_vmem, d_vmem, ts_vmem, td_vmem):
            @pl.loop(0, SW, step=16)
            def _(j):
                sv = s_vmem[0, pl.ds(j, 16)]
                dv = d_vmem[0, pl.ds(j, 16)]
                ts_vmem[0, pl.ds(j, 16)] = plsc.load_gather(ntile, [sv])
                td_vmem[0, pl.ds(j, 16)] = plsc.load_gather(ntile, [dv])

        pltpu.emit_pipeline(
            body,
            grid=(E // SW,),
            in_specs=[
                pl.BlockSpec((1, SW), lambda i: (0, i)),
                pl.BlockSpec((1, SW), lambda i: (0, i)),
            ],
            out_specs=[
                pl.BlockSpec((1, SW), lambda i: (0, i)),
                pl.BlockSpec((1, SW), lambda i: (0, i)),
            ],
            core_axis_name=("c", "s"),
            dimension_semantics=(pltpu.PARALLEL,),
        )(s_hbm, d_hbm, o_hbm.at[0], o_hbm.at[1])

    return k(nidx_pad, esrc2, edst2)


def _sc_scatter2(vals, idx2):
    C = vals.shape[2]
    Ec = vals.shape[1]
    idx3 = idx2.reshape(2, 1, Ec)
    rows_per = NPAD // 16

    @functools.partial(
        pl.kernel,
        out_type=jax.ShapeDtypeStruct((2, NPAD, C), vals.dtype),
        mesh=_sc_mesh(),
        scratch_types=[
            pltpu.VMEM_SHARED((NPAD, C), vals.dtype),
            pltpu.VMEM((ZR, C), vals.dtype),
        ],
    )
    def k(v_hbm, i_hbm, o_hbm, acc, zbuf):
        c = jax.lax.axis_index("c")
        s = jax.lax.axis_index("s")

        @pl.loop(0, ZR)
        def _(r):
            @pl.loop(0, C, step=16)
            def _(j):
                zbuf.at[pl.ds(r, 1), pl.ds(j, 16)][...] = jnp.zeros(
                    (1, 16), vals.dtype)

        @pl.loop(0, rows_per, step=ZR)
        def _(r0):
            pltpu.sync_copy(zbuf, acc.at[pl.ds(s * rows_per + r0, ZR)])

        plsc.subcore_barrier()

        def body(v_vmem, i_vmem):
            pltpu.sync_copy(v_vmem, acc.at[i_vmem.at[0]], add=True)

        pltpu.emit_pipeline(
            body,
            grid=(Ec // SW,),
            in_specs=[
                pl.BlockSpec((SW, C), lambda i: (i, 0)),
                pl.BlockSpec((1, SW), lambda i: (0, i)),
            ],
            out_specs=[],
            core_axis_name=("s",),
            dimension_semantics=(pltpu.PARALLEL,),
        )(v_hbm.at[c], i_hbm.at[c])

        plsc.subcore_barrier()

        @pl.loop(0, rows_per, step=ZR)
        def _(r0):
            pltpu.sync_copy(acc.at[pl.ds(s * rows_per + r0, ZR)],
                            o_hbm.at[c].at[pl.ds(s * rows_per + r0, ZR)])

    return k(vals, idx3)


def _edge_mlp_body(yall, ohe, gext, lin1t, out):
    ys = yall[0]
    yd = yall[1]
    pre = jnp.dot(ohe[...], gext[...], preferred_element_type=jnp.float32)

    W = pre * jax.nn.sigmoid(pre)
    g = W * (ys - yd)
    a = W * (ys + yd) * 0.5
    dxe = jnp.concatenate([g, a, g * a, g * g, a * a], axis=1)
    z = jnp.tanh(dxe)
    lt = lin1t[...]
    t = jnp.dot(z.astype(jnp.bfloat16), lt, preferred_element_type=jnp.float32)
    t = t - jnp.mean(t, axis=1, keepdims=True)
    t = t * jax.lax.rsqrt(jnp.sum(t * t, axis=1, keepdims=True) + 1e-3)
    z2 = jnp.tanh(t)
    t2 = jnp.dot(z2.astype(jnp.bfloat16), lt, preferred_element_type=jnp.float32)
    z3 = jnp.tanh(t2)
    p = W * z3[:, :DL]
    q = (0.5 * W) * (z3[:, DL:2 * DL] + z3[:, 2 * DL:3 * DL]
                     + z3[:, 3 * DL:4 * DL] + z3[:, 4 * DL:])
    out[0] = p + q
    out[1] = q - p


def _edge_mlp(yall3, ohe, gext, lin1t):
    Ec = yall3.shape[1]
    nblk = Ec // BE
    e3 = lambda e: (0, e, 0)
    wb = lambda e: (0, 0)
    return pl.pallas_call(
        _edge_mlp_body,
        grid=(nblk,),
        compiler_params=pltpu.CompilerParams(
            dimension_semantics=("parallel",)),
        in_specs=[
            pl.BlockSpec((2, BE, DL), e3),
            pl.BlockSpec((BE, DL), lambda e: (e, 0)),
            pl.BlockSpec((DL, DL), wb),
            pl.BlockSpec((FD, FD), wb),
        ],
        out_specs=pl.BlockSpec((2, BE, DL), e3),
        out_shape=jax.ShapeDtypeStruct((2, Ec, DL), jnp.float32),
    )(yall3, ohe, gext, lin1t)


def _uplift_body(x, wlt, o):
    o[...] = jnp.dot(x[...], wlt[...], preferred_element_type=jnp.float32)


def _uplift(x, wlt):
    bn = 2000
    return pl.pallas_call(
        _uplift_body,
        grid=(N // bn,),
        in_specs=[
            pl.BlockSpec((bn, x.shape[1]), lambda i: (i, 0)),
            pl.BlockSpec(wlt.shape, lambda i: (0, 0)),
        ],
        out_specs=pl.BlockSpec((bn, DL), lambda i: (i, 0)),
        out_shape=jax.ShapeDtypeStruct((N, DL), jnp.float32),
    )(x, wlt)


def _update_body(y, yold, spa, spb, dt, o):
    o[...] = (2.0 * y[...] - yold[...]
              - dt[...] * ((spa[0] + spa[1]) + (spb[0] + spb[1])))


def _update(y, yold, spa, spb, dt):
    bn = 2000
    return pl.pallas_call(
        _update_body,
        grid=(N // bn,),
        in_specs=[
            pl.BlockSpec((bn, DL), lambda i: (i, 0)),
            pl.BlockSpec((bn, DL), lambda i: (i, 0)),
            pl.BlockSpec((2, bn, DL), lambda i: (0, i, 0)),
            pl.BlockSpec((2, bn, DL), lambda i: (0, i, 0)),
            pl.BlockSpec((1, 1), lambda i: (0, 0)),
        ],
        out_specs=pl.BlockSpec((bn, DL), lambda i: (i, 0)),
        out_shape=jax.ShapeDtypeStruct((N, DL), jnp.float32),
    )(y, yold, spa, spb, dt)


def _project_body(y, wl, o):
    o[...] = jnp.dot(y[...], wl[...], preferred_element_type=jnp.float32)


def _project(y, wlpad):
    bn = 2000
    return pl.pallas_call(
        _project_body,
        grid=(N // bn,),
        in_specs=[
            pl.BlockSpec((bn, DL), lambda i: (i, 0)),
            pl.BlockSpec((DL, DL), lambda i: (0, 0)),
        ],
        out_specs=pl.BlockSpec((bn, DL), lambda i: (i, 0)),
        out_shape=jax.ShapeDtypeStruct((N, DL), jnp.float32),
    )(y, wlpad)


def kernel(x, batch, node_attr, edge_src, edge_dst, wstatic, W_lin,
           embed_table, h, fc1_w, fc1_b, lin1_w):
    L = fc1_w.shape[0]
    emb = embed_table.shape[1]

    nt = embed_table.shape[0]
    nidx = jnp.min(node_attr, axis=-1).astype(jnp.int32)
    nidx_pad = jnp.concatenate([nidx, jnp.zeros((NPAD - N,), jnp.int32)])
    esrc = edge_src.astype(jnp.int32)
    edst = edge_dst.astype(jnp.int32)
    tsd = _sc_types(nidx_pad, esrc.reshape(1, E), edst.reshape(1, E))
    ioh = jnp.arange(32, dtype=jnp.int32)[None, :]
    ohe = jnp.concatenate([
        (tsd[0, 0][:, None] == ioh).astype(jnp.bfloat16),
        (tsd[1, 0][:, None] == ioh).astype(jnp.bfloat16),
        wstatic[:, None].astype(jnp.bfloat16),
        jnp.zeros((E, 63), jnp.bfloat16),
    ], axis=1)

    EH = E // 2
    chunks = []
    for s0, s1 in ((0, EH), (EH, E)):
        chunks.append((
            jnp.concatenate([esrc[s0:s1], edst[s0:s1]]),
            jnp.stack([edst[s0:s1], esrc[s0:s1]]),
            s0, s1,
        ))

    dts = jnp.clip(h * h, 1e-4, 0.1)

    y = _uplift(x, W_lin.T)
    y_old = y

    for i in range(L):
        f1t = fc1_w[i].T
        gext = jnp.zeros((DL, DL), jnp.float32)
        gext = gext.at[:nt].set(embed_table @ f1t[:emb] + fc1_b[i][None, :])
        gext = gext.at[32:32 + nt].set(embed_table @ f1t[emb:2 * emb])
        gext = gext.at[64:65].set(f1t[2 * emb:2 * emb + 1])
        gext = gext.astype(jnp.bfloat16)
        lin1t = lin1_w[i].T.astype(jnp.bfloat16)

        sps = []
        for g_idx, s_idx, s0, s1 in chunks:
            yall3 = _sc_gather(y, g_idx).reshape(2, s1 - s0, DL)
            av = _edge_mlp(yall3, ohe[s0:s1], gext, lin1t)
            sps.append(_sc_scatter2(av, s_idx))
        dt = dts[i].reshape(1, 1)
        y_new = _update(y, y_old, sps[0], sps[1], dt)
        y_old = y
        y = y_new

    wlpad = jnp.zeros((DL, DL), jnp.float32).at[:, :W_lin.shape[1]].set(W_lin)
    x_out = _project(y, wlpad)[:, :W_lin.shape[1]]
    reg = jnp.asarray(0.0, dtype=jnp.float32)
    return x_out, reg
